# initial kernel scaffold (unmeasured)
import jax
import jax.numpy as jnp
from jax import lax
from jax.experimental import pallas as pl
from jax.experimental.pallas import tpu as pltpu


def kernel(
    x,
):
    def body(*refs):
        pass

    out_shape = jax.ShapeDtypeStruct(..., jnp.float32)
    return pl.pallas_call(body, out_shape=out_shape)(...)


# baseline (device time: 7756 ns/iter reference)
import jax
import jax.numpy as jnp
from jax import lax
from jax.experimental import pallas as pl
from jax.experimental.pallas import tpu as pltpu

N_DEV = 4


def kernel(x):
    m, n = x.shape

    def body(x_ref, out_ref, halo_ref, send_sems, recv_sems):
        my = lax.axis_index("i")
        left = lax.rem(my + N_DEV - 1, N_DEV)
        right = lax.rem(my + 1, N_DEV)

        barrier_sem = pltpu.get_barrier_semaphore()
        for nbr in (left, right):
            pl.semaphore_signal(
                barrier_sem, inc=1,
                device_id=(nbr,), device_id_type=pl.DeviceIdType.MESH,
            )
        pl.semaphore_wait(barrier_sem, 2)

        send_up = pltpu.make_async_remote_copy(
            src_ref=x_ref.at[pl.ds(0, 1), :],
            dst_ref=halo_ref.at[1],
            send_sem=send_sems.at[0],
            recv_sem=recv_sems.at[1],
            device_id=(left,),
            device_id_type=pl.DeviceIdType.MESH,
        )
        send_dn = pltpu.make_async_remote_copy(
            src_ref=x_ref.at[pl.ds(m - 1, 1), :],
            dst_ref=halo_ref.at[0],
            send_sem=send_sems.at[1],
            recv_sem=recv_sems.at[0],
            device_id=(right,),
            device_id_type=pl.DeviceIdType.MESH,
        )
        send_up.start()
        send_dn.start()

        out_ref[pl.ds(1, m - 2), :] = (
            0.25 * x_ref[pl.ds(0, m - 2), :]
            + 0.5 * x_ref[pl.ds(1, m - 2), :]
            + 0.25 * x_ref[pl.ds(2, m - 2), :]
        )

        send_up.wait()
        send_dn.wait()

        out_ref[pl.ds(0, 1), :] = (
            0.25 * halo_ref[0]
            + 0.5 * x_ref[pl.ds(0, 1), :]
            + 0.25 * x_ref[pl.ds(1, 1), :]
        )
        out_ref[pl.ds(m - 1, 1), :] = (
            0.25 * x_ref[pl.ds(m - 2, 1), :]
            + 0.5 * x_ref[pl.ds(m - 1, 1), :]
            + 0.25 * halo_ref[1]
        )

        @pl.when(my == 0)
        def _():
            out_ref[pl.ds(0, 1), :] = x_ref[pl.ds(0, 1), :]

        @pl.when(my == N_DEV - 1)
        def _():
            out_ref[pl.ds(m - 1, 1), :] = x_ref[pl.ds(m - 1, 1), :]

    return pl.pallas_call(
        body,
        out_shape=jax.ShapeDtypeStruct((m, n), x.dtype),
        in_specs=[pl.BlockSpec(memory_space=pltpu.VMEM)],
        out_specs=pl.BlockSpec(memory_space=pltpu.VMEM),
        scratch_shapes=[
            pltpu.VMEM((2, 1, n), x.dtype),
            pltpu.SemaphoreType.DMA((2,)),
            pltpu.SemaphoreType.DMA((2,)),
        ],
        compiler_params=pltpu.CompilerParams(collective_id=0),
    )(x)


# device time: 7737 ns/iter; 1.0025x vs baseline; 1.0025x over previous
import jax
import jax.numpy as jnp
from jax import lax
from jax.experimental import pallas as pl
from jax.experimental.pallas import tpu as pltpu

N_DEV = 4


def kernel(x):
    m, n = x.shape

    def body(x_ref, out_ref, halo_ref, send_sems, recv_sems):
        my = lax.axis_index("i")
        left = lax.rem(my + N_DEV - 1, N_DEV)
        right = lax.rem(my + 1, N_DEV)

        barrier_sem = pltpu.get_barrier_semaphore()
        for nbr in (left, right):
            pl.semaphore_signal(
                barrier_sem, inc=1,
                device_id=(nbr,), device_id_type=pl.DeviceIdType.MESH,
            )
        pl.semaphore_wait(barrier_sem, 2)

        send_up = pltpu.make_async_remote_copy(
            src_ref=x_ref.at[pl.ds(0, 1), :],
            dst_ref=halo_ref.at[1],
            send_sem=send_sems.at[0],
            recv_sem=recv_sems.at[1],
            device_id=(left,),
            device_id_type=pl.DeviceIdType.MESH,
        )
        send_dn = pltpu.make_async_remote_copy(
            src_ref=x_ref.at[pl.ds(m - 1, 1), :],
            dst_ref=halo_ref.at[0],
            send_sem=send_sems.at[1],
            recv_sem=recv_sems.at[0],
            device_id=(right,),
            device_id_type=pl.DeviceIdType.MESH,
        )
        send_up.start()
        send_dn.start()

        xv = x_ref[:, :]
        up = pltpu.roll(xv, 1, 0)
        dn = pltpu.roll(xv, m - 1, 0)
        out_ref[:, :] = 0.25 * (up + dn) + 0.5 * xv

        send_up.wait()
        send_dn.wait()

        out_ref[pl.ds(0, 1), :] = (
            0.25 * halo_ref[0]
            + 0.5 * x_ref[pl.ds(0, 1), :]
            + 0.25 * x_ref[pl.ds(1, 1), :]
        )
        out_ref[pl.ds(m - 1, 1), :] = (
            0.25 * x_ref[pl.ds(m - 2, 1), :]
            + 0.5 * x_ref[pl.ds(m - 1, 1), :]
            + 0.25 * halo_ref[1]
        )

        @pl.when(my == 0)
        def _():
            out_ref[pl.ds(0, 1), :] = x_ref[pl.ds(0, 1), :]

        @pl.when(my == N_DEV - 1)
        def _():
            out_ref[pl.ds(m - 1, 1), :] = x_ref[pl.ds(m - 1, 1), :]

    return pl.pallas_call(
        body,
        out_shape=jax.ShapeDtypeStruct((m, n), x.dtype),
        in_specs=[pl.BlockSpec(memory_space=pltpu.VMEM)],
        out_specs=pl.BlockSpec(memory_space=pltpu.VMEM),
        scratch_shapes=[
            pltpu.VMEM((2, 1, n), x.dtype),
            pltpu.SemaphoreType.DMA((2,)),
            pltpu.SemaphoreType.DMA((2,)),
        ],
        compiler_params=pltpu.CompilerParams(collective_id=0),
    )(x)


# device time: 7735 ns/iter; 1.0027x vs baseline; 1.0003x over previous
import jax
import jax.numpy as jnp
from jax import lax
from jax.experimental import pallas as pl
from jax.experimental.pallas import tpu as pltpu

N_DEV = 4


def kernel(x):
    m, n = x.shape

    def body(x_hbm, out_hbm, xv, ov, halo_ref,
             in_sem, out_sems, send_sems, recv_sems):
        my = lax.axis_index("i")
        left = lax.rem(my + N_DEV - 1, N_DEV)
        right = lax.rem(my + 1, N_DEV)

        barrier_sem = pltpu.get_barrier_semaphore()
        for nbr in (left, right):
            pl.semaphore_signal(
                barrier_sem, inc=1,
                device_id=(nbr,), device_id_type=pl.DeviceIdType.MESH,
            )
        in_dma = pltpu.make_async_copy(x_hbm, xv, in_sem)
        in_dma.start()
        pl.semaphore_wait(barrier_sem, 2)

        send_up = pltpu.make_async_remote_copy(
            src_ref=x_hbm.at[pl.ds(0, 1), :],
            dst_ref=halo_ref.at[1],
            send_sem=send_sems.at[0],
            recv_sem=recv_sems.at[1],
            device_id=(left,),
            device_id_type=pl.DeviceIdType.MESH,
        )
        send_dn = pltpu.make_async_remote_copy(
            src_ref=x_hbm.at[pl.ds(m - 1, 1), :],
            dst_ref=halo_ref.at[0],
            send_sem=send_sems.at[1],
            recv_sem=recv_sems.at[0],
            device_id=(right,),
            device_id_type=pl.DeviceIdType.MESH,
        )
        send_up.start()
        send_dn.start()

        in_dma.wait()
        xvv = xv[:, :]
        up = pltpu.roll(xvv, 1, 0)
        dn = pltpu.roll(xvv, m - 1, 0)
        ov[:, :] = 0.25 * (up + dn) + 0.5 * xvv

        bulk = pltpu.make_async_copy(
            ov.at[pl.ds(8, m - 16), :],
            out_hbm.at[pl.ds(8, m - 16), :],
            out_sems.at[0],
        )
        bulk.start()

        send_up.wait()
        send_dn.wait()

        ov[pl.ds(0, 1), :] = (
            0.25 * halo_ref[0]
            + 0.5 * xv[pl.ds(0, 1), :]
            + 0.25 * xv[pl.ds(1, 1), :]
        )
        ov[pl.ds(m - 1, 1), :] = (
            0.25 * xv[pl.ds(m - 2, 1), :]
            + 0.5 * xv[pl.ds(m - 1, 1), :]
            + 0.25 * halo_ref[1]
        )

        @pl.when(my == 0)
        def _():
            ov[pl.ds(0, 1), :] = xv[pl.ds(0, 1), :]

        @pl.when(my == N_DEV - 1)
        def _():
            ov[pl.ds(m - 1, 1), :] = xv[pl.ds(m - 1, 1), :]

        edge_top = pltpu.make_async_copy(
            ov.at[pl.ds(0, 8), :], out_hbm.at[pl.ds(0, 8), :], out_sems.at[1]
        )
        edge_bot = pltpu.make_async_copy(
            ov.at[pl.ds(m - 8, 8), :],
            out_hbm.at[pl.ds(m - 8, 8), :],
            out_sems.at[2],
        )
        edge_top.start()
        edge_bot.start()

        bulk.wait()
        edge_top.wait()
        edge_bot.wait()

    return pl.pallas_call(
        body,
        out_shape=jax.ShapeDtypeStruct((m, n), x.dtype),
        in_specs=[pl.BlockSpec(memory_space=pl.ANY)],
        out_specs=pl.BlockSpec(memory_space=pl.ANY),
        scratch_shapes=[
            pltpu.VMEM((m, n), x.dtype),
            pltpu.VMEM((m, n), x.dtype),
            pltpu.VMEM((2, 1, n), x.dtype),
            pltpu.SemaphoreType.DMA,
            pltpu.SemaphoreType.DMA((3,)),
            pltpu.SemaphoreType.DMA((2,)),
            pltpu.SemaphoreType.DMA((2,)),
        ],
        compiler_params=pltpu.CompilerParams(collective_id=0),
    )(x)
